# baseline (speedup 1.0000x reference)
import functools

import jax
import jax.numpy as jnp
from jax import lax
from jax.experimental import pallas as pl
from jax.experimental.pallas import tpu as pltpu
from jax.experimental.pallas import tpu_sc as plsc

L = 16
NC = 2
NS = 16
NW = NC * NS

_GDN = lax.GatherDimensionNumbers(
    offset_dims=(), collapsed_slice_dims=(0,), start_index_map=(0,))


def _perm(x, idx):
    return lax.gather(x, idx[:, None], _GDN, (1,),
                      mode=lax.GatherScatterMode.PROMISE_IN_BOUNDS)


def _spcl_sc(zw, ei, gt, sm, lam):
    E = ei.shape[1]
    Dw = zw.shape[1]
    nwc = Dw // L
    epw = E // NW
    B = 400
    SG = 80
    nchunks = epw // B

    mesh = plsc.VectorSubcoreMesh(core_axis_name="c", subcore_axis_name="s")

    @functools.partial(
        pl.kernel,
        mesh=mesh,
        out_type=jax.ShapeDtypeStruct((NW, L), jnp.float32),
        compiler_params=pltpu.CompilerParams(needs_layout_passes=False, use_tc_tiling_on_sc=False),
        scratch_types=[
            pltpu.VMEM((2 * B,), jnp.int32),
            pltpu.VMEM((2 * B,), jnp.int32),
            pltpu.VMEM((2 * B, Dw), jnp.float32),
            pltpu.VMEM((2 * B, Dw), jnp.float32),
            pltpu.VMEM((2 * B,), jnp.float32),
            pltpu.VMEM((2 * B,), jnp.float32),
            pltpu.VMEM((L,), jnp.float32),
            pltpu.VMEM((L,), jnp.float32),
            pltpu.SemaphoreType.DMA,
            pltpu.SemaphoreType.DMA,
        ],
    )
    def k(z_h, ei_h, gt_h, sm_h, lam_h, out_h,
          sidx, didx, srows, drows, gtv, wv, lamv, outv, sem_i, sem_r):
        wid = lax.axis_index("c") * NS + lax.axis_index("s")
        pltpu.sync_copy(lam_h, lamv)
        lam_vec = lamv[...]
        lane = lax.iota(jnp.int32, L)
        perms = {w: lane ^ w for w in (8, 4, 2, 1)}

        def issue_idx(ci, buf):
            base = pl.multiple_of(wid * epw + ci * B, 8)
            bo = pl.multiple_of(buf * B, 8)
            pltpu.async_copy(ei_h.at[0, pl.ds(base, B)],
                             sidx.at[pl.ds(bo, B)], sem_i)
            pltpu.async_copy(ei_h.at[1, pl.ds(base, B)],
                             didx.at[pl.ds(bo, B)], sem_i)
            pltpu.async_copy(gt_h.at[pl.ds(base, B)],
                             gtv.at[pl.ds(bo, B)], sem_i)
            pltpu.async_copy(sm_h.at[pl.ds(base, B)],
                             wv.at[pl.ds(bo, B)], sem_i)

        def wait_idx(buf):
            bo = pl.multiple_of(buf * B, 8)
            pltpu.make_async_copy(ei_h.at[0, pl.ds(0, B)],
                                  sidx.at[pl.ds(bo, B)], sem_i).wait()
            pltpu.make_async_copy(ei_h.at[1, pl.ds(0, B)],
                                  didx.at[pl.ds(bo, B)], sem_i).wait()
            pltpu.make_async_copy(gt_h.at[pl.ds(0, B)],
                                  gtv.at[pl.ds(bo, B)], sem_i).wait()
            pltpu.make_async_copy(sm_h.at[pl.ds(0, B)],
                                  wv.at[pl.ds(bo, B)], sem_i).wait()

        def issue_rows(buf):
            for j in range(B // SG):
                s = pl.ds(pl.multiple_of(buf * B + j * SG, 8), SG)
                pltpu.async_copy(z_h.at[sidx.at[s]], srows.at[s], sem_r)
                pltpu.async_copy(z_h.at[didx.at[s]], drows.at[s], sem_r)

        def wait_rows(buf):
            for j in range(B // SG):
                s = pl.ds(pl.multiple_of(buf * B + j * SG, 8), SG)
                pltpu.make_async_copy(z_h.at[sidx.at[s]], srows.at[s],
                                      sem_r).wait()
                pltpu.make_async_copy(z_h.at[didx.at[s]], drows.at[s],
                                      sem_r).wait()

        issue_idx(0, 0)
        wait_idx(0)
        issue_rows(0)
        issue_idx(1, 1)

        def chunk_body(c, tot):
            buf = lax.rem(c, 2)
            nbuf = 1 - buf
            wait_rows(buf)

            @pl.when(c + 1 < nchunks)
            def _():
                wait_idx(nbuf)
                issue_rows(nbuf)

            bo = pl.multiple_of(buf * B, 8)

            def edge_body(e, acc):
                ew = lax.rem(e, L)
                gb = pl.multiple_of(bo + e - ew, 8)
                row = bo + e
                ps = []
                for f in range(nwc):
                    a = plsc.bitcast(srows[row, pl.ds(f * L, L)],
                                     jnp.bfloat16)
                    b = plsc.bitcast(drows[row, pl.ds(f * L, L)],
                                     jnp.bfloat16)
                    ps.append(a * b)
                n = nwc
                while n > 1:
                    ps = [ps[2 * i] + ps[2 * i + 1]
                          for i in range(n // 2)] + ps[n & ~1:]
                    n = (n + 1) // 2
                ev, od = plsc.unpack(ps[0], format=plsc.PackFormat.INTERLEAVED,
                                     preferred_element_type=jnp.float32)
                h = ev + od
                for w in (8, 4, 2, 1):
                    h = h + _perm(h, perms[w])
                p = 1.0 / (1.0 + jnp.exp(-h))
                diff = p - gtv[pl.ds(gb, L)]
                cont = wv[pl.ds(gb, L)] * (diff * diff - lam_vec)
                return acc + jnp.where(lane == ew, cont, 0.0)

            tot = plsc.parallel_loop(0, B, unroll=4, carry=tot)(edge_body)

            @pl.when(c + 2 < nchunks)
            def _():
                issue_idx(c + 2, buf)

            return tot

        tot = lax.fori_loop(0, nchunks, chunk_body,
                            jnp.zeros((L,), jnp.float32))
        outv[...] = tot
        pltpu.sync_copy(outv, out_h.at[wid])

    return k(zw, ei, gt, sm, lam)


def kernel(z, edge_index, _lambda, gt_edge, s_mask):
    zi = lax.bitcast_convert_type(z, jnp.int32)
    zr = zi + 0x7FFF + ((zi >> 16) & 1)
    hi = zr >> 16
    packed = ((hi[:, 1::2] << 16) | (hi[:, 0::2] & 0xFFFF))
    zw = lax.bitcast_convert_type(packed, jnp.float32)
    ei = edge_index.astype(jnp.int32)
    lam = jnp.full((L,), _lambda, jnp.float32)
    parts = _spcl_sc(zw, ei,
                     gt_edge.astype(jnp.float32),
                     s_mask.astype(jnp.float32), lam)
    return jnp.sum(parts)


# speedup vs baseline: 2.1875x; 2.1875x over previous
import functools

import jax
import jax.numpy as jnp
from jax import lax
from jax.experimental import pallas as pl
from jax.experimental.pallas import tpu as pltpu
from jax.experimental.pallas import tpu_sc as plsc

L = 16
NC = 2
NS = 16
NW = NC * NS

_GDN = lax.GatherDimensionNumbers(
    offset_dims=(), collapsed_slice_dims=(0,), start_index_map=(0,))


def _perm(x, idx):
    return lax.gather(x, idx[:, None], _GDN, (1,),
                      mode=lax.GatherScatterMode.PROMISE_IN_BOUNDS)


def _spcl_sc(zw, ei, gt, sm, lam):
    E = ei.shape[1]
    Dw = zw.shape[1]
    nwc = Dw // L
    epw = E // NW
    B = 400
    SG = 80
    nchunks = epw // B

    mesh = plsc.VectorSubcoreMesh(core_axis_name="c", subcore_axis_name="s")

    @functools.partial(
        pl.kernel,
        mesh=mesh,
        out_type=jax.ShapeDtypeStruct((NW, L), jnp.float32),
        compiler_params=pltpu.CompilerParams(needs_layout_passes=False, use_tc_tiling_on_sc=False),
        scratch_types=[
            pltpu.VMEM((2 * B,), jnp.int32),
            pltpu.VMEM((2 * B,), jnp.int32),
            pltpu.VMEM((2 * B, Dw), jnp.float32),
            pltpu.VMEM((2 * B, Dw), jnp.float32),
            pltpu.VMEM((2 * B,), jnp.float32),
            pltpu.VMEM((2 * B,), jnp.float32),
            pltpu.VMEM((L,), jnp.float32),
            pltpu.VMEM((L,), jnp.float32),
            pltpu.SemaphoreType.DMA,
            pltpu.SemaphoreType.DMA,
        ],
    )
    def k(z_h, ei_h, gt_h, sm_h, lam_h, out_h,
          sidx, didx, srows, drows, gtv, wv, lamv, outv, sem_i, sem_r):
        wid = lax.axis_index("c") * NS + lax.axis_index("s")
        pltpu.sync_copy(lam_h, lamv)
        lam_vec = lamv[...]
        lane = lax.iota(jnp.int32, L)
        perms = {w: lane ^ w for w in (8, 4, 2, 1)}

        def issue_idx(ci, buf):
            base = pl.multiple_of(wid * epw + ci * B, 8)
            bo = pl.multiple_of(buf * B, 8)
            pltpu.async_copy(ei_h.at[0, pl.ds(base, B)],
                             sidx.at[pl.ds(bo, B)], sem_i)
            pltpu.async_copy(ei_h.at[1, pl.ds(base, B)],
                             didx.at[pl.ds(bo, B)], sem_i)
            pltpu.async_copy(gt_h.at[pl.ds(base, B)],
                             gtv.at[pl.ds(bo, B)], sem_i)
            pltpu.async_copy(sm_h.at[pl.ds(base, B)],
                             wv.at[pl.ds(bo, B)], sem_i)

        def wait_idx(buf):
            bo = pl.multiple_of(buf * B, 8)
            pltpu.make_async_copy(ei_h.at[0, pl.ds(0, B)],
                                  sidx.at[pl.ds(bo, B)], sem_i).wait()
            pltpu.make_async_copy(ei_h.at[1, pl.ds(0, B)],
                                  didx.at[pl.ds(bo, B)], sem_i).wait()
            pltpu.make_async_copy(gt_h.at[pl.ds(0, B)],
                                  gtv.at[pl.ds(bo, B)], sem_i).wait()
            pltpu.make_async_copy(sm_h.at[pl.ds(0, B)],
                                  wv.at[pl.ds(bo, B)], sem_i).wait()

        def issue_rows(buf):
            for j in range(B // SG):
                s = pl.ds(pl.multiple_of(buf * B + j * SG, 8), SG)
                pltpu.async_copy(z_h.at[sidx.at[s]], srows.at[s], sem_r)
                pltpu.async_copy(z_h.at[didx.at[s]], drows.at[s], sem_r)

        def wait_rows(buf):
            for j in range(B // SG):
                s = pl.ds(pl.multiple_of(buf * B + j * SG, 8), SG)
                pltpu.make_async_copy(z_h.at[sidx.at[s]], srows.at[s],
                                      sem_r).wait()
                pltpu.make_async_copy(z_h.at[didx.at[s]], drows.at[s],
                                      sem_r).wait()

        issue_idx(0, 0)
        wait_idx(0)
        issue_rows(0)
        issue_idx(1, 1)

        def chunk_body(c, tot):
            buf = lax.rem(c, 2)
            nbuf = 1 - buf
            wait_rows(buf)

            @pl.when(c + 1 < nchunks)
            def _():
                wait_idx(nbuf)
                issue_rows(nbuf)

            bo = pl.multiple_of(buf * B, 8)

            def edge_body(e, acc):
                ew = lax.rem(e, L)
                gb = pl.multiple_of(bo + e - ew, 8)
                row = bo + e
                ps = []
                for f in range(nwc):
                    a = plsc.bitcast(srows[row, pl.ds(f * L, L)],
                                     jnp.bfloat16)
                    b = plsc.bitcast(drows[row, pl.ds(f * L, L)],
                                     jnp.bfloat16)
                    ps.append(a * b)
                n = nwc
                while n > 1:
                    ps = [ps[2 * i] + ps[2 * i + 1]
                          for i in range(n // 2)] + ps[n & ~1:]
                    n = (n + 1) // 2
                ev, od = plsc.unpack(ps[0], format=plsc.PackFormat.INTERLEAVED,
                                     preferred_element_type=jnp.float32)
                h = ev + od
                for w in (8, 4, 2, 1):
                    h = h + _perm(h, perms[w])
                p = 1.0 / (1.0 + jnp.exp(-h))
                diff = p - gtv[pl.ds(gb, L)]
                cont = wv[pl.ds(gb, L)] * (diff * diff - lam_vec)
                return acc + jnp.where(lane == ew, cont, 0.0)

            tot = plsc.parallel_loop(0, B, unroll=4, carry=tot)(edge_body)

            @pl.when(c + 2 < nchunks)
            def _():
                issue_idx(c + 2, buf)

            return tot

        tot = lax.fori_loop(0, nchunks, chunk_body,
                            jnp.zeros((L,), jnp.float32))
        outv[...] = tot
        pltpu.sync_copy(outv, out_h.at[wid])

    return k(zw, ei, gt, sm, lam)


def kernel(z, edge_index, _lambda, gt_edge, s_mask):
    n, d = z.shape
    zw = lax.bitcast_convert_type(
        z.astype(jnp.bfloat16).reshape(n, d // 2, 2), jnp.float32)
    ei = edge_index.astype(jnp.int32)
    lam = jnp.full((L,), _lambda, jnp.float32)
    parts = _spcl_sc(zw, ei,
                     gt_edge.astype(jnp.float32),
                     s_mask.astype(jnp.float32), lam)
    return jnp.sum(parts)


# speedup vs baseline: 2.3135x; 1.0576x over previous
import functools

import jax
import jax.numpy as jnp
from jax import lax
from jax.experimental import pallas as pl
from jax.experimental.pallas import tpu as pltpu
from jax.experimental.pallas import tpu_sc as plsc

L = 16
NC = 2
NS = 16
NW = NC * NS

_GDN = lax.GatherDimensionNumbers(
    offset_dims=(), collapsed_slice_dims=(0,), start_index_map=(0,))


def _perm(x, idx):
    return lax.gather(x, idx[:, None], _GDN, (1,),
                      mode=lax.GatherScatterMode.PROMISE_IN_BOUNDS)


def _spcl_sc(z, ei, gt, sm, lam):
    E = ei.shape[1]
    N, D = z.shape
    Dw = D // 2
    nwc = Dw // L
    RB = 25
    rpt = N // NS
    epw = E // NW
    B = 400
    SG = 80
    nchunks = epw // B

    mesh = plsc.VectorSubcoreMesh(core_axis_name="c", subcore_axis_name="s")

    @functools.partial(
        pl.kernel,
        mesh=mesh,
        out_type=(jax.ShapeDtypeStruct((NW, L), jnp.float32),
                  jax.ShapeDtypeStruct((NC, N, Dw), jnp.float32)),
        compiler_params=pltpu.CompilerParams(needs_layout_passes=False, use_tc_tiling_on_sc=False),
        scratch_types=[
            pltpu.VMEM((2 * B,), jnp.int32),
            pltpu.VMEM((2 * B,), jnp.int32),
            pltpu.VMEM((2 * B, Dw), jnp.float32),
            pltpu.VMEM((2 * B, Dw), jnp.float32),
            pltpu.VMEM((2 * B,), jnp.float32),
            pltpu.VMEM((2 * B,), jnp.float32),
            pltpu.VMEM((L,), jnp.float32),
            pltpu.VMEM((L,), jnp.float32),
            pltpu.VMEM((RB, 2 * Dw), jnp.float32),
            pltpu.VMEM((RB, Dw), jnp.float32),
            pltpu.SemaphoreType.DMA,
            pltpu.SemaphoreType.DMA,
        ],
    )
    def k(z_h, ei_h, gt_h, sm_h, lam_h, out_h, slab_h,
          sidx, didx, srows, drows, gtv, wv, lamv, outv,
          ztmp, pbuf, sem_i, sem_r):
        cidx = lax.axis_index("c")
        sid = lax.axis_index("s")
        wid = cidx * NS + sid
        myslab = slab_h.at[cidx]

        def pack_block(b, _):
            rb = pl.multiple_of(sid * rpt + b * RB, 1)
            pltpu.sync_copy(z_h.at[pl.ds(rb, RB)], ztmp)

            def pack_row(r):
                for kk in range(Dw // L):
                    a = ztmp[r, pl.ds(kk * 2 * L, L)]
                    bvec = ztmp[r, pl.ds(kk * 2 * L + L, L)]
                    w = plsc.bitcast(
                        plsc.pack(a, bvec,
                                  format=plsc.PackFormat.INTERLEAVED),
                        jnp.float32)
                    pbuf[r, pl.ds(kk * L, L)] = w

            plsc.parallel_loop(0, RB, unroll=1)(pack_row)
            pltpu.sync_copy(pbuf, myslab.at[pl.ds(rb, RB)])
            return _

        lax.fori_loop(0, rpt // RB, pack_block, 0)
        plsc.subcore_barrier()

        pltpu.sync_copy(lam_h, lamv)
        lam_vec = lamv[...]
        lane = lax.iota(jnp.int32, L)
        perms = {w: lane ^ w for w in (8, 4, 2, 1)}

        def issue_idx(ci, buf):
            base = pl.multiple_of(wid * epw + ci * B, 8)
            bo = pl.multiple_of(buf * B, 8)
            pltpu.async_copy(ei_h.at[0, pl.ds(base, B)],
                             sidx.at[pl.ds(bo, B)], sem_i)
            pltpu.async_copy(ei_h.at[1, pl.ds(base, B)],
                             didx.at[pl.ds(bo, B)], sem_i)
            pltpu.async_copy(gt_h.at[pl.ds(base, B)],
                             gtv.at[pl.ds(bo, B)], sem_i)
            pltpu.async_copy(sm_h.at[pl.ds(base, B)],
                             wv.at[pl.ds(bo, B)], sem_i)

        def wait_idx(buf):
            bo = pl.multiple_of(buf * B, 8)
            pltpu.make_async_copy(ei_h.at[0, pl.ds(0, B)],
                                  sidx.at[pl.ds(bo, B)], sem_i).wait()
            pltpu.make_async_copy(ei_h.at[1, pl.ds(0, B)],
                                  didx.at[pl.ds(bo, B)], sem_i).wait()
            pltpu.make_async_copy(gt_h.at[pl.ds(0, B)],
                                  gtv.at[pl.ds(bo, B)], sem_i).wait()
            pltpu.make_async_copy(sm_h.at[pl.ds(0, B)],
                                  wv.at[pl.ds(bo, B)], sem_i).wait()

        def issue_rows(buf):
            for j in range(B // SG):
                s = pl.ds(pl.multiple_of(buf * B + j * SG, 8), SG)
                pltpu.async_copy(myslab.at[sidx.at[s]], srows.at[s], sem_r)
                pltpu.async_copy(myslab.at[didx.at[s]], drows.at[s], sem_r)

        def wait_rows(buf):
            for j in range(B // SG):
                s = pl.ds(pl.multiple_of(buf * B + j * SG, 8), SG)
                pltpu.make_async_copy(myslab.at[sidx.at[s]], srows.at[s],
                                      sem_r).wait()
                pltpu.make_async_copy(myslab.at[didx.at[s]], drows.at[s],
                                      sem_r).wait()

        issue_idx(0, 0)
        wait_idx(0)
        issue_rows(0)
        issue_idx(1, 1)

        def chunk_body(c, tot):
            buf = lax.rem(c, 2)
            nbuf = 1 - buf
            wait_rows(buf)

            @pl.when(c + 1 < nchunks)
            def _():
                wait_idx(nbuf)
                issue_rows(nbuf)

            bo = pl.multiple_of(buf * B, 8)

            def edge_body(e, acc):
                ew = lax.rem(e, L)
                gb = pl.multiple_of(bo + e - ew, 8)
                row = bo + e
                ps = []
                for f in range(nwc):
                    a = plsc.bitcast(srows[row, pl.ds(f * L, L)],
                                     jnp.bfloat16)
                    b = plsc.bitcast(drows[row, pl.ds(f * L, L)],
                                     jnp.bfloat16)
                    ps.append(a * b)
                n = nwc
                while n > 1:
                    ps = [ps[2 * i] + ps[2 * i + 1]
                          for i in range(n // 2)] + ps[n & ~1:]
                    n = (n + 1) // 2
                ev, od = plsc.unpack(ps[0], format=plsc.PackFormat.INTERLEAVED,
                                     preferred_element_type=jnp.float32)
                h = ev + od
                for w in (8, 4, 2, 1):
                    h = h + _perm(h, perms[w])
                p = 1.0 / (1.0 + jnp.exp(-h))
                diff = p - gtv[pl.ds(gb, L)]
                cont = wv[pl.ds(gb, L)] * (diff * diff - lam_vec)
                return acc + jnp.where(lane == ew, cont, 0.0)

            tot = plsc.parallel_loop(0, B, unroll=4, carry=tot)(edge_body)

            @pl.when(c + 2 < nchunks)
            def _():
                issue_idx(c + 2, buf)

            return tot

        tot = lax.fori_loop(0, nchunks, chunk_body,
                            jnp.zeros((L,), jnp.float32))
        outv[...] = tot
        pltpu.sync_copy(outv, out_h.at[wid])

    return k(z, ei, gt, sm, lam)


def kernel(z, edge_index, _lambda, gt_edge, s_mask):
    ei = edge_index.astype(jnp.int32)
    lam = jnp.full((L,), _lambda, jnp.float32)
    parts, _ = _spcl_sc(z, ei,
                        gt_edge.astype(jnp.float32),
                        s_mask.astype(jnp.float32), lam)
    return jnp.sum(parts)


# speedup vs baseline: 2.5798x; 1.1151x over previous
import functools

import jax
import jax.numpy as jnp
from jax import lax
from jax.experimental import pallas as pl
from jax.experimental.pallas import tpu as pltpu
from jax.experimental.pallas import tpu_sc as plsc

L = 16
NC = 2
NS = 16
NW = NC * NS

_GDN = lax.GatherDimensionNumbers(
    offset_dims=(), collapsed_slice_dims=(0,), start_index_map=(0,))


def _perm(x, idx):
    return lax.gather(x, idx[:, None], _GDN, (1,),
                      mode=lax.GatherScatterMode.PROMISE_IN_BOUNDS)


def _spcl_sc(z, ei, gt, sm, lam):
    E = ei.shape[1]
    N, D = z.shape
    Dw = D // 2
    nwc = Dw // L
    RB = 125
    rpt = N // NS
    epw = E // NW
    B = 400
    SG = 80
    nchunks = epw // B

    mesh = plsc.VectorSubcoreMesh(core_axis_name="c", subcore_axis_name="s")

    @functools.partial(
        pl.kernel,
        mesh=mesh,
        out_type=(jax.ShapeDtypeStruct((NW, L), jnp.float32),
                  jax.ShapeDtypeStruct((NC, N, Dw), jnp.float32)),
        compiler_params=pltpu.CompilerParams(needs_layout_passes=False, use_tc_tiling_on_sc=False),
        scratch_types=[
            pltpu.VMEM((2 * B,), jnp.int32),
            pltpu.VMEM((2 * B,), jnp.int32),
            pltpu.VMEM((2 * B, Dw), jnp.float32),
            pltpu.VMEM((2 * B, Dw), jnp.float32),
            pltpu.VMEM((2 * B,), jnp.float32),
            pltpu.VMEM((2 * B,), jnp.float32),
            pltpu.VMEM((L,), jnp.float32),
            pltpu.VMEM((L,), jnp.float32),
            pltpu.VMEM((RB, 2 * Dw), jnp.float32),
            pltpu.VMEM((RB, Dw), jnp.float32),
            pltpu.SemaphoreType.DMA,
            pltpu.SemaphoreType.DMA,
        ],
    )
    def k(z_h, ei_h, gt_h, sm_h, lam_h, out_h, slab_h,
          sidx, didx, srows, drows, gtv, wv, lamv, outv,
          ztmp, pbuf, sem_i, sem_r):
        cidx = lax.axis_index("c")
        sid = lax.axis_index("s")
        wid = cidx * NS + sid
        myslab = slab_h.at[cidx]

        def pack_block(b, _):
            rb = pl.multiple_of(sid * rpt + b * RB, 1)
            pltpu.sync_copy(z_h.at[pl.ds(rb, RB)], ztmp)

            def pack_row(r):
                for kk in range(Dw // L):
                    a = ztmp[r, pl.ds(kk * 2 * L, L)]
                    bvec = ztmp[r, pl.ds(kk * 2 * L + L, L)]
                    w = plsc.bitcast(
                        plsc.pack(a, bvec,
                                  format=plsc.PackFormat.INTERLEAVED),
                        jnp.float32)
                    pbuf[r, pl.ds(kk * L, L)] = w

            plsc.parallel_loop(0, RB, unroll=1)(pack_row)
            pltpu.sync_copy(pbuf, myslab.at[pl.ds(rb, RB)])
            return _

        lax.fori_loop(0, rpt // RB, pack_block, 0)
        plsc.subcore_barrier()

        pltpu.sync_copy(lam_h, lamv)
        lam_vec = lamv[...]
        lane = lax.iota(jnp.int32, L)
        perms = {w: lane ^ w for w in (8, 4, 2, 1)}

        def issue_idx(ci, buf):
            base = pl.multiple_of(wid * epw + ci * B, 8)
            bo = pl.multiple_of(buf * B, 8)
            pltpu.async_copy(ei_h.at[0, pl.ds(base, B)],
                             sidx.at[pl.ds(bo, B)], sem_i)
            pltpu.async_copy(ei_h.at[1, pl.ds(base, B)],
                             didx.at[pl.ds(bo, B)], sem_i)
            pltpu.async_copy(gt_h.at[pl.ds(base, B)],
                             gtv.at[pl.ds(bo, B)], sem_i)
            pltpu.async_copy(sm_h.at[pl.ds(base, B)],
                             wv.at[pl.ds(bo, B)], sem_i)

        def wait_idx(buf):
            bo = pl.multiple_of(buf * B, 8)
            pltpu.make_async_copy(ei_h.at[0, pl.ds(0, B)],
                                  sidx.at[pl.ds(bo, B)], sem_i).wait()
            pltpu.make_async_copy(ei_h.at[1, pl.ds(0, B)],
                                  didx.at[pl.ds(bo, B)], sem_i).wait()
            pltpu.make_async_copy(gt_h.at[pl.ds(0, B)],
                                  gtv.at[pl.ds(bo, B)], sem_i).wait()
            pltpu.make_async_copy(sm_h.at[pl.ds(0, B)],
                                  wv.at[pl.ds(bo, B)], sem_i).wait()

        def issue_rows(buf):
            for j in range(B // SG):
                s = pl.ds(pl.multiple_of(buf * B + j * SG, 8), SG)
                pltpu.async_copy(myslab.at[sidx.at[s]], srows.at[s], sem_r)
                pltpu.async_copy(myslab.at[didx.at[s]], drows.at[s], sem_r)

        def wait_rows(buf):
            for j in range(B // SG):
                s = pl.ds(pl.multiple_of(buf * B + j * SG, 8), SG)
                pltpu.make_async_copy(myslab.at[sidx.at[s]], srows.at[s],
                                      sem_r).wait()
                pltpu.make_async_copy(myslab.at[didx.at[s]], drows.at[s],
                                      sem_r).wait()

        issue_idx(0, 0)
        wait_idx(0)
        issue_rows(0)
        issue_idx(1, 1)

        def chunk_body(c, tot):
            buf = lax.rem(c, 2)
            nbuf = 1 - buf
            wait_rows(buf)

            @pl.when(c + 1 < nchunks)
            def _():
                wait_idx(nbuf)
                issue_rows(nbuf)

            bo = pl.multiple_of(buf * B, 8)

            def edge_body(e, acc):
                ew = lax.rem(e, L)
                gb = pl.multiple_of(bo + e - ew, 8)
                row = bo + e
                ps = []
                for f in range(nwc):
                    a = plsc.bitcast(srows[row, pl.ds(f * L, L)],
                                     jnp.bfloat16)
                    b = plsc.bitcast(drows[row, pl.ds(f * L, L)],
                                     jnp.bfloat16)
                    ps.append(a * b)
                n = nwc
                while n > 1:
                    ps = [ps[2 * i] + ps[2 * i + 1]
                          for i in range(n // 2)] + ps[n & ~1:]
                    n = (n + 1) // 2
                ev, od = plsc.unpack(ps[0], format=plsc.PackFormat.INTERLEAVED,
                                     preferred_element_type=jnp.float32)
                h = ev + od
                for w in (8, 4, 2, 1):
                    h = h + _perm(h, perms[w])
                p = 1.0 / (1.0 + jnp.exp(-h))
                diff = p - gtv[pl.ds(gb, L)]
                cont = wv[pl.ds(gb, L)] * (diff * diff - lam_vec)
                return acc + jnp.where(lane == ew, cont, 0.0)

            tot = plsc.parallel_loop(0, B, unroll=4, carry=tot)(edge_body)

            @pl.when(c + 2 < nchunks)
            def _():
                issue_idx(c + 2, buf)

            return tot

        tot = lax.fori_loop(0, nchunks, chunk_body,
                            jnp.zeros((L,), jnp.float32))
        outv[...] = tot
        pltpu.sync_copy(outv, out_h.at[wid])

    return k(z, ei, gt, sm, lam)


def kernel(z, edge_index, _lambda, gt_edge, s_mask):
    ei = edge_index.astype(jnp.int32)
    lam = jnp.full((L,), _lambda, jnp.float32)
    parts, _ = _spcl_sc(z, ei,
                        gt_edge.astype(jnp.float32),
                        s_mask.astype(jnp.float32), lam)
    return jnp.sum(parts)
